# Initial kernel scaffold; baseline (speedup 1.0000x reference)
import jax
import jax.numpy as jnp
from jax.experimental import pallas as pl


def kernel(inputs):
    raise NotImplementedError("write your pallas kernel here")


# speedup vs baseline: 8.5298x; 8.5298x over previous
import functools

import jax
import jax.numpy as jnp
from jax import lax
from jax.experimental import pallas as pl
from jax.experimental.pallas import tpu as pltpu
from jax.experimental.pallas import tpu_sc as plsc

B, H, W, C = 16, 512, 512, 3
N = H * W
NB = 4096
CHUNK = 8192
LANES = 16

_POS_LO = 0.10 * (N - 1)
_POS_HI = 0.90 * (N - 1)
K_LO = int(_POS_LO)
K_HI = int(_POS_HI)
FRAC_LO = _POS_LO - K_LO
FRAC_HI = _POS_HI - K_HI


def _mean_minmax_kernel(x_ref, m_ref, mn_ref, mx_ref):
    x = x_ref[0]
    j = lax.broadcasted_iota(jnp.int32, (W * C, W), 0)
    p = lax.broadcasted_iota(jnp.int32, (W * C, W), 1)
    wmat = jnp.where((j // 3) == p, jnp.float32(1.0 / 3.0), jnp.float32(0.0))
    m = jnp.dot(x, wmat, preferred_element_type=jnp.float32,
                precision=lax.Precision.HIGHEST)
    m_ref[0] = m
    mn_ref[0] = jnp.full((1, 128), jnp.min(m), jnp.float32)
    mx_ref[0] = jnp.full((1, 128), jnp.max(m), jnp.float32)


_mean_call = pl.pallas_call(
    _mean_minmax_kernel,
    grid=(B,),
    in_specs=[pl.BlockSpec((1, H, W * C), lambda i: (i, 0, 0))],
    out_specs=[
        pl.BlockSpec((1, H, W), lambda i: (i, 0, 0)),
        pl.BlockSpec((1, 1, 128), lambda i: (i, 0, 0)),
        pl.BlockSpec((1, 1, 128), lambda i: (i, 0, 0)),
    ],
    out_shape=[
        jax.ShapeDtypeStruct((B, H, W), jnp.float32),
        jax.ShapeDtypeStruct((B, 1, 128), jnp.float32),
        jax.ShapeDtypeStruct((B, 1, 128), jnp.float32),
    ],
)


def _sc_body(means_hbm, mn_hbm, mx_hbm, lo_hbm, up_hbm,
             buf, hist, cum, mnv, mxv, row_lo, row_up):
    c = lax.axis_index("c")
    s = lax.axis_index("s")

    @pl.when(s < 8)
    def _():
        sample = c * 8 + s
        pltpu.sync_copy(mn_hbm.at[sample], mnv)
        pltpu.sync_copy(mx_hbm.at[sample], mxv)
        mn = mnv[...]
        mx = mxv[...]
        rng = mx - mn
        inv_w = jnp.float32(NB) / jnp.maximum(rng, jnp.float32(1e-30))
        w1 = rng * jnp.float32(1.0 / NB)

        def zero_body(i, _):
            hist[pl.ds(i * LANES, LANES)] = jnp.zeros((LANES,), jnp.int32)
            return 0

        lax.fori_loop(0, NB // LANES, zero_body, 0)

        ones = jnp.ones((LANES,), jnp.int32)

        def chunk_body(ci, _):
            pltpu.sync_copy(means_hbm.at[sample, pl.ds(ci * CHUNK, CHUNK)], buf)

            def inner(i, _):
                v = buf[pl.ds(i * LANES, LANES)]
                idx = jnp.minimum(((v - mn) * inv_w).astype(jnp.int32), NB - 1)
                plsc.addupdate_scatter(hist, [idx], ones)
                return 0

            lax.fori_loop(0, CHUNK // LANES, inner, 0)
            return 0

        lax.fori_loop(0, N // CHUNK, chunk_body, 0)

        def cum_body(i, carry):
            hv = hist[pl.ds(i * LANES, LANES)]
            cum[pl.ds(i * LANES, LANES)] = carry + plsc.cumsum(hv)
            return carry + jnp.sum(hv)

        lax.fori_loop(0, NB // LANES, cum_body, jnp.zeros((LANES,), jnp.int32))

        def order_stat(k):
            def b_body(i, acc):
                cv = cum[pl.ds(i * LANES, LANES)]
                return acc + plsc.all_reduce_population_count(cv <= k)

            b = lax.fori_loop(0, NB // LANES, b_body,
                              jnp.zeros((LANES,), jnp.int32))
            cnt = plsc.load_gather(hist, [b])
            below = plsc.load_gather(cum, [b]) - cnt
            rank = (jnp.float32(k) - below.astype(jnp.float32)
                    + jnp.float32(0.5)) / cnt.astype(jnp.float32)
            return mn + w1 * (b.astype(jnp.float32) + rank)

        v_lo0 = order_stat(K_LO)
        v_lo1 = order_stat(K_LO + 1)
        v_hi0 = order_stat(K_HI)
        v_hi1 = order_stat(K_HI + 1)
        lower = v_lo0 + jnp.float32(FRAC_LO) * (v_lo1 - v_lo0)
        upper = v_hi0 + jnp.float32(FRAC_HI) * (v_hi1 - v_hi0)
        row_lo[...] = lower
        row_up[...] = upper
        pltpu.sync_copy(row_lo, lo_hbm.at[sample])
        pltpu.sync_copy(row_up, up_hbm.at[sample])


@functools.cache
def _sc_quantiles_call():
    return functools.partial(
        pl.kernel,
        out_type=[
            jax.ShapeDtypeStruct((B, LANES), jnp.float32),
            jax.ShapeDtypeStruct((B, LANES), jnp.float32),
        ],
        mesh=plsc.VectorSubcoreMesh(core_axis_name="c", subcore_axis_name="s",
                                    num_cores=2, num_subcores=16),
        compiler_params=pltpu.CompilerParams(needs_layout_passes=False),
        scratch_types=[
            pltpu.VMEM((CHUNK,), jnp.float32),
            pltpu.VMEM((NB,), jnp.int32),
            pltpu.VMEM((NB,), jnp.int32),
            pltpu.VMEM((LANES,), jnp.float32),
            pltpu.VMEM((LANES,), jnp.float32),
            pltpu.VMEM((LANES,), jnp.float32),
            pltpu.VMEM((LANES,), jnp.float32),
        ],
    )(_sc_body)


def _norm_kernel(lo_ref, up_ref, m_ref, o_ref):
    lo = lo_ref[0, 0, 0]
    up = up_ref[0, 0, 0]
    rng = jnp.maximum(up - lo, jnp.float32(1e-6))
    o_ref[0] = jnp.clip((m_ref[0] - lo) / rng, 0.0, 1.0)


_norm_call = pl.pallas_call(
    _norm_kernel,
    grid=(B,),
    in_specs=[
        pl.BlockSpec((1, 1, 1), lambda i: (i, 0, 0), memory_space=pltpu.SMEM),
        pl.BlockSpec((1, 1, 1), lambda i: (i, 0, 0), memory_space=pltpu.SMEM),
        pl.BlockSpec((1, H, W), lambda i: (i, 0, 0)),
    ],
    out_specs=pl.BlockSpec((1, H, W), lambda i: (i, 0, 0)),
    out_shape=jax.ShapeDtypeStruct((B, H, W), jnp.float32),
)


def kernel(inputs):
    x = inputs.reshape(B, H, W * C)
    means, mn, mx = _mean_call(x)
    mn16 = mn.reshape(B, 128)[:, :LANES]
    mx16 = mx.reshape(B, 128)[:, :LANES]
    lo, up = _sc_quantiles_call()(means.reshape(B, N), mn16, mx16)
    out = _norm_call(lo[:, :1].reshape(B, 1, 1), up[:, :1].reshape(B, 1, 1), means)
    return out.reshape(B, H, W, 1)


# speedup vs baseline: 8.5439x; 1.0017x over previous
import functools

import jax
import jax.numpy as jnp
from jax import lax
from jax.experimental import pallas as pl
from jax.experimental.pallas import tpu as pltpu
from jax.experimental.pallas import tpu_sc as plsc

B, H, W, C = 16, 512, 512, 3
N = H * W
NB = 4096
CHUNK = 8192
LANES = 16

_POS_LO = 0.10 * (N - 1)
_POS_HI = 0.90 * (N - 1)
K_LO = int(_POS_LO)
K_HI = int(_POS_HI)
FRAC_LO = _POS_LO - K_LO
FRAC_HI = _POS_HI - K_HI


def _mean_minmax_kernel(x_ref, m_ref, mn_ref, mx_ref):
    x = x_ref[0]
    j = lax.broadcasted_iota(jnp.int32, (W * C, W), 0)
    p = lax.broadcasted_iota(jnp.int32, (W * C, W), 1)
    wmat = jnp.where((j // 3) == p, jnp.float32(1.0 / 3.0), jnp.float32(0.0))
    m = jnp.dot(x, wmat, preferred_element_type=jnp.float32,
                precision=lax.Precision.HIGHEST)
    m_ref[0] = m
    mn_ref[0] = jnp.full((1, 128), jnp.min(m), jnp.float32)
    mx_ref[0] = jnp.full((1, 128), jnp.max(m), jnp.float32)


_mean_call = pl.pallas_call(
    _mean_minmax_kernel,
    grid=(B,),
    in_specs=[pl.BlockSpec((1, H, W * C), lambda i: (i, 0, 0))],
    out_specs=[
        pl.BlockSpec((1, H, W), lambda i: (i, 0, 0)),
        pl.BlockSpec((1, 1, 128), lambda i: (i, 0, 0)),
        pl.BlockSpec((1, 1, 128), lambda i: (i, 0, 0)),
    ],
    out_shape=[
        jax.ShapeDtypeStruct((B, H, W), jnp.float32),
        jax.ShapeDtypeStruct((B, 1, 128), jnp.float32),
        jax.ShapeDtypeStruct((B, 1, 128), jnp.float32),
    ],
)


def _sc_body(means_hbm, mn_hbm, mx_hbm, lo_hbm, up_hbm,
             buf, hist, cum, mnv, mxv, row_lo, row_up):
    c = lax.axis_index("c")
    s = lax.axis_index("s")

    @pl.when(s < 8)
    def _():
        sample = c * 8 + s
        pltpu.sync_copy(mn_hbm.at[sample, 0, pl.ds(0, LANES)], mnv)
        pltpu.sync_copy(mx_hbm.at[sample, 0, pl.ds(0, LANES)], mxv)
        mn = mnv[...]
        mx = mxv[...]
        rng = mx - mn
        inv_w = jnp.float32(NB) / jnp.maximum(rng, jnp.float32(1e-30))
        w1 = rng * jnp.float32(1.0 / NB)

        def zero_body(i, _):
            hist[pl.ds(i * LANES, LANES)] = jnp.zeros((LANES,), jnp.int32)
            return 0

        lax.fori_loop(0, NB // LANES, zero_body, 0)

        ones = jnp.ones((LANES,), jnp.int32)

        def chunk_body(ci, _):
            pltpu.sync_copy(means_hbm.at[sample, pl.ds(ci * CHUNK, CHUNK)], buf)

            def inner(i, _):
                v = buf[pl.ds(i * LANES, LANES)]
                idx = jnp.minimum(((v - mn) * inv_w).astype(jnp.int32), NB - 1)
                plsc.addupdate_scatter(hist, [idx], ones)
                return 0

            lax.fori_loop(0, CHUNK // LANES, inner, 0)
            return 0

        lax.fori_loop(0, N // CHUNK, chunk_body, 0)

        def cum_body(i, carry):
            hv = hist[pl.ds(i * LANES, LANES)]
            cum[pl.ds(i * LANES, LANES)] = carry + plsc.cumsum(hv)
            return carry + jnp.sum(hv)

        lax.fori_loop(0, NB // LANES, cum_body, jnp.zeros((LANES,), jnp.int32))

        def order_stat(k):
            def b_body(i, acc):
                cv = cum[pl.ds(i * LANES, LANES)]
                return acc + plsc.all_reduce_population_count(cv <= k)

            b = lax.fori_loop(0, NB // LANES, b_body,
                              jnp.zeros((LANES,), jnp.int32))
            cnt = plsc.load_gather(hist, [b])
            below = plsc.load_gather(cum, [b]) - cnt
            rank = (jnp.float32(k) - below.astype(jnp.float32)
                    + jnp.float32(0.5)) / cnt.astype(jnp.float32)
            return mn + w1 * (b.astype(jnp.float32) + rank)

        v_lo0 = order_stat(K_LO)
        v_lo1 = order_stat(K_LO + 1)
        v_hi0 = order_stat(K_HI)
        v_hi1 = order_stat(K_HI + 1)
        lower = v_lo0 + jnp.float32(FRAC_LO) * (v_lo1 - v_lo0)
        upper = v_hi0 + jnp.float32(FRAC_HI) * (v_hi1 - v_hi0)
        row_lo[...] = lower
        row_up[...] = upper
        pltpu.sync_copy(row_lo, lo_hbm.at[sample, 0, pl.ds(0, LANES)])
        pltpu.sync_copy(row_up, up_hbm.at[sample, 0, pl.ds(0, LANES)])


@functools.cache
def _sc_quantiles_call():
    return functools.partial(
        pl.kernel,
        out_type=[
            jax.ShapeDtypeStruct((B, 1, 128), jnp.float32),
            jax.ShapeDtypeStruct((B, 1, 128), jnp.float32),
        ],
        mesh=plsc.VectorSubcoreMesh(core_axis_name="c", subcore_axis_name="s",
                                    num_cores=2, num_subcores=16),
        compiler_params=pltpu.CompilerParams(needs_layout_passes=False),
        scratch_types=[
            pltpu.VMEM((CHUNK,), jnp.float32),
            pltpu.VMEM((NB,), jnp.int32),
            pltpu.VMEM((NB,), jnp.int32),
            pltpu.VMEM((LANES,), jnp.float32),
            pltpu.VMEM((LANES,), jnp.float32),
            pltpu.VMEM((LANES,), jnp.float32),
            pltpu.VMEM((LANES,), jnp.float32),
        ],
    )(_sc_body)


def _norm_kernel(lo_ref, up_ref, m_ref, o_ref):
    lo = lo_ref[0, 0, 0]
    up = up_ref[0, 0, 0]
    rng = jnp.maximum(up - lo, jnp.float32(1e-6))
    o_ref[0] = jnp.clip((m_ref[0] - lo) / rng, 0.0, 1.0)


_norm_call = pl.pallas_call(
    _norm_kernel,
    grid=(B,),
    in_specs=[
        pl.BlockSpec((1, 1, 128), lambda i: (i, 0, 0), memory_space=pltpu.SMEM),
        pl.BlockSpec((1, 1, 128), lambda i: (i, 0, 0), memory_space=pltpu.SMEM),
        pl.BlockSpec((1, H, W), lambda i: (i, 0, 0)),
    ],
    out_specs=pl.BlockSpec((1, H, W), lambda i: (i, 0, 0)),
    out_shape=jax.ShapeDtypeStruct((B, H, W), jnp.float32),
)


def kernel(inputs):
    x = inputs.reshape(B, H, W * C)
    means, mn, mx = _mean_call(x)
    lo, up = _sc_quantiles_call()(means.reshape(B, N), mn, mx)
    out = _norm_call(lo, up, means)
    return out.reshape(B, H, W, 1)


# speedup vs baseline: 11.2153x; 1.3127x over previous
import functools

import jax
import jax.numpy as jnp
from jax import lax
from jax.experimental import pallas as pl
from jax.experimental.pallas import tpu as pltpu
from jax.experimental.pallas import tpu_sc as plsc

B, H, W, C = 16, 512, 512, 3
N = H * W
NB = 4096
CHUNK = 8192
LANES = 16

_POS_LO = 0.10 * (N - 1)
_POS_HI = 0.90 * (N - 1)
K_LO = int(_POS_LO)
K_HI = int(_POS_HI)
FRAC_LO = _POS_LO - K_LO
FRAC_HI = _POS_HI - K_HI


def _mean_minmax_kernel(x_ref, m_ref, mn_ref, mx_ref):
    x = x_ref[0]
    j = lax.broadcasted_iota(jnp.int32, (W * C, W), 0)
    p = lax.broadcasted_iota(jnp.int32, (W * C, W), 1)
    wmat = jnp.where((j // 3) == p, jnp.float32(1.0 / 3.0), jnp.float32(0.0))
    m = jnp.dot(x, wmat, preferred_element_type=jnp.float32,
                precision=lax.Precision.HIGHEST)
    m_ref[0] = m
    mn_ref[0] = jnp.full((1, 128), jnp.min(m), jnp.float32)
    mx_ref[0] = jnp.full((1, 128), jnp.max(m), jnp.float32)


_mean_call = pl.pallas_call(
    _mean_minmax_kernel,
    grid=(B,),
    in_specs=[pl.BlockSpec((1, H, W * C), lambda i: (i, 0, 0))],
    out_specs=[
        pl.BlockSpec((1, H, W), lambda i: (i, 0, 0)),
        pl.BlockSpec((1, 1, 128), lambda i: (i, 0, 0)),
        pl.BlockSpec((1, 1, 128), lambda i: (i, 0, 0)),
    ],
    out_shape=[
        jax.ShapeDtypeStruct((B, H, W), jnp.float32),
        jax.ShapeDtypeStruct((B, 1, 128), jnp.float32),
        jax.ShapeDtypeStruct((B, 1, 128), jnp.float32),
    ],
)


def _sc_body(means_hbm, mn_hbm, mx_hbm, lo_hbm, up_hbm, hist_hbm,
             buf, hist, cum, part, mnv, mxv, row_lo, row_up):
    c = lax.axis_index("c")
    s = lax.axis_index("s")
    sample = c * 8 + lax.rem(s, 8)
    half = lax.div(s, 8)
    wid = c * 16 + s
    partner = c * 16 + lax.rem(s + 8, 16)

    pltpu.sync_copy(mn_hbm.at[sample, 0, pl.ds(0, LANES)], mnv)
    pltpu.sync_copy(mx_hbm.at[sample, 0, pl.ds(0, LANES)], mxv)
    mn = mnv[...]
    mx = mxv[...]
    rng = mx - mn
    inv_w = jnp.float32(NB) / jnp.maximum(rng, jnp.float32(1e-30))
    w1 = rng * jnp.float32(1.0 / NB)

    def zero_body(i, _):
        hist[pl.ds(i * LANES, LANES)] = jnp.zeros((LANES,), jnp.int32)
        return 0

    lax.fori_loop(0, NB // LANES, zero_body, 0)

    ones = jnp.ones((LANES,), jnp.int32)
    base = half * (N // 2)

    def chunk_body(ci, _):
        pltpu.sync_copy(
            means_hbm.at[sample, pl.ds(base + ci * CHUNK, CHUNK)], buf)

        def inner(i, _):
            for u in range(4):
                v = buf[pl.ds(i * (4 * LANES) + u * LANES, LANES)]
                idx = jnp.minimum(((v - mn) * inv_w).astype(jnp.int32), NB - 1)
                plsc.addupdate_scatter(hist, [idx], ones)
            return 0

        lax.fori_loop(0, CHUNK // (4 * LANES), inner, 0)
        return 0

    lax.fori_loop(0, (N // 2) // CHUNK, chunk_body, 0)

    pltpu.sync_copy(hist, hist_hbm.at[wid])
    plsc.subcore_barrier()
    pltpu.sync_copy(hist_hbm.at[partner], part)

    def merge_body(i, _):
        sl = pl.ds(i * LANES, LANES)
        hist[sl] = hist[sl] + part[sl]
        return 0

    lax.fori_loop(0, NB // LANES, merge_body, 0)

    def cum_body(i, carry):
        hv = hist[pl.ds(i * LANES, LANES)]
        cum[pl.ds(i * LANES, LANES)] = carry + plsc.cumsum(hv)
        return carry + jnp.sum(hv)

    lax.fori_loop(0, NB // LANES, cum_body, jnp.zeros((LANES,), jnp.int32))

    def order_stat(k):
        def b_body(i, acc):
            cv = cum[pl.ds(i * LANES, LANES)]
            return acc + plsc.all_reduce_population_count(cv <= k)

        b = lax.fori_loop(0, NB // LANES, b_body,
                          jnp.zeros((LANES,), jnp.int32))
        cnt = plsc.load_gather(hist, [b])
        below = plsc.load_gather(cum, [b]) - cnt
        rank = (jnp.float32(k) - below.astype(jnp.float32)
                + jnp.float32(0.5)) / cnt.astype(jnp.float32)
        return mn + w1 * (b.astype(jnp.float32) + rank)

    @pl.when(half == 0)
    def _():
        v_lo0 = order_stat(K_LO)
        v_lo1 = order_stat(K_LO + 1)
        v_hi0 = order_stat(K_HI)
        v_hi1 = order_stat(K_HI + 1)
        lower = v_lo0 + jnp.float32(FRAC_LO) * (v_lo1 - v_lo0)
        upper = v_hi0 + jnp.float32(FRAC_HI) * (v_hi1 - v_hi0)
        row_lo[...] = lower
        row_up[...] = upper
        pltpu.sync_copy(row_lo, lo_hbm.at[sample, 0, pl.ds(0, LANES)])
        pltpu.sync_copy(row_up, up_hbm.at[sample, 0, pl.ds(0, LANES)])


@functools.cache
def _sc_quantiles_call():
    return functools.partial(
        pl.kernel,
        out_type=[
            jax.ShapeDtypeStruct((B, 1, 128), jnp.float32),
            jax.ShapeDtypeStruct((B, 1, 128), jnp.float32),
            jax.ShapeDtypeStruct((32, NB), jnp.int32),
        ],
        mesh=plsc.VectorSubcoreMesh(core_axis_name="c", subcore_axis_name="s",
                                    num_cores=2, num_subcores=16),
        compiler_params=pltpu.CompilerParams(needs_layout_passes=False),
        scratch_types=[
            pltpu.VMEM((CHUNK,), jnp.float32),
            pltpu.VMEM((NB,), jnp.int32),
            pltpu.VMEM((NB,), jnp.int32),
            pltpu.VMEM((NB,), jnp.int32),
            pltpu.VMEM((LANES,), jnp.float32),
            pltpu.VMEM((LANES,), jnp.float32),
            pltpu.VMEM((LANES,), jnp.float32),
            pltpu.VMEM((LANES,), jnp.float32),
        ],
    )(_sc_body)


def _norm_kernel(lo_ref, up_ref, m_ref, o_ref):
    lo = lo_ref[0, 0, 0]
    up = up_ref[0, 0, 0]
    rng = jnp.maximum(up - lo, jnp.float32(1e-6))
    o_ref[0] = jnp.clip((m_ref[0] - lo) / rng, 0.0, 1.0)


_norm_call = pl.pallas_call(
    _norm_kernel,
    grid=(B,),
    in_specs=[
        pl.BlockSpec((1, 1, 128), lambda i: (i, 0, 0), memory_space=pltpu.SMEM),
        pl.BlockSpec((1, 1, 128), lambda i: (i, 0, 0), memory_space=pltpu.SMEM),
        pl.BlockSpec((1, H, W), lambda i: (i, 0, 0)),
    ],
    out_specs=pl.BlockSpec((1, H, W), lambda i: (i, 0, 0)),
    out_shape=jax.ShapeDtypeStruct((B, H, W), jnp.float32),
)


def kernel(inputs):
    x = inputs.reshape(B, H, W * C)
    means, mn, mx = _mean_call(x)
    lo, up, _ = _sc_quantiles_call()(means.reshape(B, N), mn, mx)
    out = _norm_call(lo, up, means)
    return out.reshape(B, H, W, 1)


# speedup vs baseline: 13.0442x; 1.1631x over previous
import functools

import jax
import jax.numpy as jnp
from jax import lax
from jax.experimental import pallas as pl
from jax.experimental.pallas import tpu as pltpu
from jax.experimental.pallas import tpu_sc as plsc

B, H, W, C = 16, 512, 512, 3
N = H * W
NB = 4096
CHUNK = 8192
LANES = 16

_POS_LO = 0.10 * (N - 1)
_POS_HI = 0.90 * (N - 1)
K_LO = int(_POS_LO)
K_HI = int(_POS_HI)
FRAC_LO = _POS_LO - K_LO
FRAC_HI = _POS_HI - K_HI


def _mean_minmax_kernel(x_ref, m_ref, mn_ref, mx_ref):
    x = x_ref[0]
    j = lax.broadcasted_iota(jnp.int32, (W * C, W), 0)
    p = lax.broadcasted_iota(jnp.int32, (W * C, W), 1)
    wmat = jnp.where((j // 3) == p, jnp.float32(1.0),
                     jnp.float32(0.0)).astype(jnp.bfloat16)
    hi = x.astype(jnp.bfloat16)
    lo = (x - hi.astype(jnp.float32)).astype(jnp.bfloat16)
    ssum = (jnp.dot(hi, wmat, preferred_element_type=jnp.float32)
            + jnp.dot(lo, wmat, preferred_element_type=jnp.float32))
    m = ssum * jnp.float32(1.0 / 3.0)
    m_ref[0] = m
    mn_ref[0] = jnp.full((1, 128), jnp.min(m), jnp.float32)
    mx_ref[0] = jnp.full((1, 128), jnp.max(m), jnp.float32)


_mean_call = pl.pallas_call(
    _mean_minmax_kernel,
    grid=(B,),
    in_specs=[pl.BlockSpec((1, H, W * C), lambda i: (i, 0, 0))],
    out_specs=[
        pl.BlockSpec((1, H, W), lambda i: (i, 0, 0)),
        pl.BlockSpec((1, 1, 128), lambda i: (i, 0, 0)),
        pl.BlockSpec((1, 1, 128), lambda i: (i, 0, 0)),
    ],
    out_shape=[
        jax.ShapeDtypeStruct((B, H, W), jnp.float32),
        jax.ShapeDtypeStruct((B, 1, 128), jnp.float32),
        jax.ShapeDtypeStruct((B, 1, 128), jnp.float32),
    ],
)


def _sc_body(means_hbm, mn_hbm, mx_hbm, lo_hbm, up_hbm, hist_hbm,
             buf, hist, cum, part, mnv, mxv, row_lo, row_up):
    c = lax.axis_index("c")
    s = lax.axis_index("s")
    sample = c * 8 + lax.rem(s, 8)
    half = lax.div(s, 8)
    wid = c * 16 + s
    partner = c * 16 + lax.rem(s + 8, 16)

    pltpu.sync_copy(mn_hbm.at[sample, 0, pl.ds(0, LANES)], mnv)
    pltpu.sync_copy(mx_hbm.at[sample, 0, pl.ds(0, LANES)], mxv)
    mn = mnv[...]
    mx = mxv[...]
    rng = mx - mn
    inv_w = jnp.float32(NB) / jnp.maximum(rng, jnp.float32(1e-30))
    w1 = rng * jnp.float32(1.0 / NB)

    def zero_body(i, _):
        hist[pl.ds(i * LANES, LANES)] = jnp.zeros((LANES,), jnp.int32)
        return 0

    lax.fori_loop(0, NB // LANES, zero_body, 0)

    ones = jnp.ones((LANES,), jnp.int32)
    base = half * (N // 2)

    def chunk_body(ci, _):
        pltpu.sync_copy(
            means_hbm.at[sample, pl.ds(base + ci * CHUNK, CHUNK)], buf)

        def inner(i, _):
            for u in range(4):
                v = buf[pl.ds(i * (4 * LANES) + u * LANES, LANES)]
                idx = jnp.minimum(((v - mn) * inv_w).astype(jnp.int32), NB - 1)
                plsc.addupdate_scatter(hist, [idx], ones)
            return 0

        lax.fori_loop(0, CHUNK // (4 * LANES), inner, 0)
        return 0

    lax.fori_loop(0, (N // 2) // CHUNK, chunk_body, 0)

    pltpu.sync_copy(hist, hist_hbm.at[wid])
    plsc.subcore_barrier()
    pltpu.sync_copy(hist_hbm.at[partner], part)

    def merge_body(i, _):
        sl = pl.ds(i * LANES, LANES)
        hist[sl] = hist[sl] + part[sl]
        return 0

    lax.fori_loop(0, NB // LANES, merge_body, 0)

    def cum_body(i, carry):
        hv = hist[pl.ds(i * LANES, LANES)]
        cum[pl.ds(i * LANES, LANES)] = carry + plsc.cumsum(hv)
        return carry + jnp.sum(hv)

    lax.fori_loop(0, NB // LANES, cum_body, jnp.zeros((LANES,), jnp.int32))

    def order_stat(k):
        def b_body(i, acc):
            cv = cum[pl.ds(i * LANES, LANES)]
            return acc + plsc.all_reduce_population_count(cv <= k)

        b = lax.fori_loop(0, NB // LANES, b_body,
                          jnp.zeros((LANES,), jnp.int32))
        cnt = plsc.load_gather(hist, [b])
        below = plsc.load_gather(cum, [b]) - cnt
        rank = (jnp.float32(k) - below.astype(jnp.float32)
                + jnp.float32(0.5)) / cnt.astype(jnp.float32)
        return mn + w1 * (b.astype(jnp.float32) + rank)

    @pl.when(half == 0)
    def _():
        v_lo0 = order_stat(K_LO)
        v_lo1 = order_stat(K_LO + 1)
        v_hi0 = order_stat(K_HI)
        v_hi1 = order_stat(K_HI + 1)
        lower = v_lo0 + jnp.float32(FRAC_LO) * (v_lo1 - v_lo0)
        upper = v_hi0 + jnp.float32(FRAC_HI) * (v_hi1 - v_hi0)
        row_lo[...] = lower
        row_up[...] = upper
        pltpu.sync_copy(row_lo, lo_hbm.at[sample, 0, pl.ds(0, LANES)])
        pltpu.sync_copy(row_up, up_hbm.at[sample, 0, pl.ds(0, LANES)])


@functools.cache
def _sc_quantiles_call():
    return functools.partial(
        pl.kernel,
        out_type=[
            jax.ShapeDtypeStruct((B, 1, 128), jnp.float32),
            jax.ShapeDtypeStruct((B, 1, 128), jnp.float32),
            jax.ShapeDtypeStruct((32, NB), jnp.int32),
        ],
        mesh=plsc.VectorSubcoreMesh(core_axis_name="c", subcore_axis_name="s",
                                    num_cores=2, num_subcores=16),
        compiler_params=pltpu.CompilerParams(needs_layout_passes=False),
        scratch_types=[
            pltpu.VMEM((CHUNK,), jnp.float32),
            pltpu.VMEM((NB,), jnp.int32),
            pltpu.VMEM((NB,), jnp.int32),
            pltpu.VMEM((NB,), jnp.int32),
            pltpu.VMEM((LANES,), jnp.float32),
            pltpu.VMEM((LANES,), jnp.float32),
            pltpu.VMEM((LANES,), jnp.float32),
            pltpu.VMEM((LANES,), jnp.float32),
        ],
    )(_sc_body)


def _norm_kernel(lo_ref, up_ref, m_ref, o_ref):
    lo = lo_ref[0, 0, 0]
    up = up_ref[0, 0, 0]
    rng = jnp.maximum(up - lo, jnp.float32(1e-6))
    o_ref[0] = jnp.clip((m_ref[0] - lo) / rng, 0.0, 1.0)


_norm_call = pl.pallas_call(
    _norm_kernel,
    grid=(B,),
    in_specs=[
        pl.BlockSpec((1, 1, 128), lambda i: (i, 0, 0), memory_space=pltpu.SMEM),
        pl.BlockSpec((1, 1, 128), lambda i: (i, 0, 0), memory_space=pltpu.SMEM),
        pl.BlockSpec((1, H, W), lambda i: (i, 0, 0)),
    ],
    out_specs=pl.BlockSpec((1, H, W), lambda i: (i, 0, 0)),
    out_shape=jax.ShapeDtypeStruct((B, H, W), jnp.float32),
)


def kernel(inputs):
    x = inputs.reshape(B, H, W * C)
    means, mn, mx = _mean_call(x)
    lo, up, _ = _sc_quantiles_call()(means.reshape(B, N), mn, mx)
    out = _norm_call(lo, up, means)
    return out.reshape(B, H, W, 1)


# speedup vs baseline: 13.8932x; 1.0651x over previous
import functools

import jax
import jax.numpy as jnp
from jax import lax
from jax.experimental import pallas as pl
from jax.experimental.pallas import tpu as pltpu
from jax.experimental.pallas import tpu_sc as plsc

B, H, W, C = 16, 512, 512, 3
N = H * W
NB = 4096
CHUNK = 8192
LANES = 16

_POS_LO = 0.10 * (N - 1)
_POS_HI = 0.90 * (N - 1)
K_LO = int(_POS_LO)
K_HI = int(_POS_HI)
FRAC_LO = _POS_LO - K_LO
FRAC_HI = _POS_HI - K_HI


def _mean_minmax_kernel(x_ref, m_ref, mn_ref, mx_ref):
    x = x_ref[0]
    j = lax.broadcasted_iota(jnp.int32, (W * C, W), 0)
    p = lax.broadcasted_iota(jnp.int32, (W * C, W), 1)
    wmat = jnp.where((j // 3) == p, jnp.float32(1.0),
                     jnp.float32(0.0)).astype(jnp.bfloat16)
    hi = x.astype(jnp.bfloat16)
    lo = (x - hi.astype(jnp.float32)).astype(jnp.bfloat16)
    ssum = (jnp.dot(hi, wmat, preferred_element_type=jnp.float32)
            + jnp.dot(lo, wmat, preferred_element_type=jnp.float32))
    m = ssum * jnp.float32(1.0 / 3.0)
    m_ref[0] = m
    mn_ref[0] = jnp.full((1, 128), jnp.min(m), jnp.float32)
    mx_ref[0] = jnp.full((1, 128), jnp.max(m), jnp.float32)


_mean_call = pl.pallas_call(
    _mean_minmax_kernel,
    grid=(B,),
    in_specs=[pl.BlockSpec((1, H, W * C), lambda i: (i, 0, 0))],
    out_specs=[
        pl.BlockSpec((1, H, W), lambda i: (i, 0, 0)),
        pl.BlockSpec((1, 1, 128), lambda i: (i, 0, 0)),
        pl.BlockSpec((1, 1, 128), lambda i: (i, 0, 0)),
    ],
    out_shape=[
        jax.ShapeDtypeStruct((B, H, W), jnp.float32),
        jax.ShapeDtypeStruct((B, 1, 128), jnp.float32),
        jax.ShapeDtypeStruct((B, 1, 128), jnp.float32),
    ],
)


def _sc_body(means_hbm, mn_hbm, mx_hbm, lo_hbm, up_hbm, hist_hbm,
             buf0, buf1, hist, cum, part, mnv, mxv, row_lo, row_up,
             sem0, sem1):
    c = lax.axis_index("c")
    s = lax.axis_index("s")
    sample = c * 8 + lax.rem(s, 8)
    half = lax.div(s, 8)
    wid = c * 16 + s
    partner = c * 16 + lax.rem(s + 8, 16)

    pltpu.sync_copy(mn_hbm.at[sample, 0, pl.ds(0, LANES)], mnv)
    pltpu.sync_copy(mx_hbm.at[sample, 0, pl.ds(0, LANES)], mxv)
    mn = mnv[...]
    mx = mxv[...]
    rng = mx - mn
    inv_w = jnp.float32(NB) / jnp.maximum(rng, jnp.float32(1e-30))
    w1 = rng * jnp.float32(1.0 / NB)

    def zero_body(i, _):
        hist[pl.ds(i * LANES, LANES)] = jnp.zeros((LANES,), jnp.int32)
        return 0

    lax.fori_loop(0, NB // LANES, zero_body, 0)

    ones = jnp.ones((LANES,), jnp.int32)
    base = half * (N // 2)
    neg_mn_scaled = -(mn * inv_w)

    def src(ci):
        return means_hbm.at[sample, pl.ds(base + ci * CHUNK, CHUNK)]

    def scan_chunk(b):
        def inner(i, _):
            for u in range(8):
                v = b[pl.ds(i * (8 * LANES) + u * LANES, LANES)]
                idx = jnp.minimum((v * inv_w + neg_mn_scaled).astype(jnp.int32),
                                  NB - 1)
                plsc.addupdate_scatter(hist, [idx], ones)
            return 0

        lax.fori_loop(0, CHUNK // (8 * LANES), inner, 0)

    npairs = (N // 2) // (2 * CHUNK)
    pltpu.async_copy(src(0), buf0, sem0)

    def pair_body(p, _):
        c0 = p * 2
        pltpu.async_copy(src(c0 + 1), buf1, sem1)
        pltpu.make_async_copy(src(c0), buf0, sem0).wait()
        scan_chunk(buf0)

        @pl.when(p < npairs - 1)
        def _():
            pltpu.async_copy(src(c0 + 2), buf0, sem0)

        pltpu.make_async_copy(src(c0 + 1), buf1, sem1).wait()
        scan_chunk(buf1)
        return 0

    lax.fori_loop(0, npairs, pair_body, 0)

    pltpu.sync_copy(hist, hist_hbm.at[wid])
    plsc.subcore_barrier()
    pltpu.sync_copy(hist_hbm.at[partner], part)

    def cum_body(i, carry):
        sl = pl.ds(i * LANES, LANES)
        hv = hist[sl] + part[sl]
        hist[sl] = hv
        cum[sl] = carry + plsc.cumsum(hv)
        return carry + jnp.sum(hv)

    lax.fori_loop(0, NB // LANES, cum_body, jnp.zeros((LANES,), jnp.int32))

    @pl.when(half == 0)
    def _():
        def b4_body(i, accs):
            cv = cum[pl.ds(i * LANES, LANES)]
            return (accs[0] + plsc.all_reduce_population_count(cv <= K_LO),
                    accs[1] + plsc.all_reduce_population_count(cv <= K_LO + 1),
                    accs[2] + plsc.all_reduce_population_count(cv <= K_HI),
                    accs[3] + plsc.all_reduce_population_count(cv <= K_HI + 1))

        z = jnp.zeros((LANES,), jnp.int32)
        b4 = lax.fori_loop(0, NB // LANES, b4_body, (z, z, z, z))

        def order_stat(k, b):
            cnt = plsc.load_gather(hist, [b])
            below = plsc.load_gather(cum, [b]) - cnt
            rank = (jnp.float32(k) - below.astype(jnp.float32)
                    + jnp.float32(0.5)) / cnt.astype(jnp.float32)
            return mn + w1 * (b.astype(jnp.float32) + rank)

        v_lo0 = order_stat(K_LO, b4[0])
        v_lo1 = order_stat(K_LO + 1, b4[1])
        v_hi0 = order_stat(K_HI, b4[2])
        v_hi1 = order_stat(K_HI + 1, b4[3])
        lower = v_lo0 + jnp.float32(FRAC_LO) * (v_lo1 - v_lo0)
        upper = v_hi0 + jnp.float32(FRAC_HI) * (v_hi1 - v_hi0)
        row_lo[...] = lower
        row_up[...] = upper
        pltpu.sync_copy(row_lo, lo_hbm.at[sample, 0, pl.ds(0, LANES)])
        pltpu.sync_copy(row_up, up_hbm.at[sample, 0, pl.ds(0, LANES)])


@functools.cache
def _sc_quantiles_call():
    return functools.partial(
        pl.kernel,
        out_type=[
            jax.ShapeDtypeStruct((B, 1, 128), jnp.float32),
            jax.ShapeDtypeStruct((B, 1, 128), jnp.float32),
            jax.ShapeDtypeStruct((32, NB), jnp.int32),
        ],
        mesh=plsc.VectorSubcoreMesh(core_axis_name="c", subcore_axis_name="s",
                                    num_cores=2, num_subcores=16),
        compiler_params=pltpu.CompilerParams(needs_layout_passes=False),
        scratch_types=[
            pltpu.VMEM((CHUNK,), jnp.float32),
            pltpu.VMEM((CHUNK,), jnp.float32),
            pltpu.VMEM((NB,), jnp.int32),
            pltpu.VMEM((NB,), jnp.int32),
            pltpu.VMEM((NB,), jnp.int32),
            pltpu.VMEM((LANES,), jnp.float32),
            pltpu.VMEM((LANES,), jnp.float32),
            pltpu.VMEM((LANES,), jnp.float32),
            pltpu.VMEM((LANES,), jnp.float32),
            pltpu.SemaphoreType.DMA,
            pltpu.SemaphoreType.DMA,
        ],
    )(_sc_body)


def _norm_kernel(lo_ref, up_ref, m_ref, o_ref):
    lo = lo_ref[0, 0, 0]
    up = up_ref[0, 0, 0]
    rng = jnp.maximum(up - lo, jnp.float32(1e-6))
    o_ref[0] = jnp.clip((m_ref[0] - lo) / rng, 0.0, 1.0)


_norm_call = pl.pallas_call(
    _norm_kernel,
    grid=(B,),
    in_specs=[
        pl.BlockSpec((1, 1, 128), lambda i: (i, 0, 0), memory_space=pltpu.SMEM),
        pl.BlockSpec((1, 1, 128), lambda i: (i, 0, 0), memory_space=pltpu.SMEM),
        pl.BlockSpec((1, H, W), lambda i: (i, 0, 0)),
    ],
    out_specs=pl.BlockSpec((1, H, W), lambda i: (i, 0, 0)),
    out_shape=jax.ShapeDtypeStruct((B, H, W), jnp.float32),
)


def kernel(inputs):
    x = inputs.reshape(B, H, W * C)
    means, mn, mx = _mean_call(x)
    lo, up, _ = _sc_quantiles_call()(means.reshape(B, N), mn, mx)
    out = _norm_call(lo, up, means)
    return out.reshape(B, H, W, 1)
